# baseline (speedup 1.0000x reference)
import functools

import jax
import jax.numpy as jnp
from jax import lax
from jax.experimental import pallas as pl
from jax.experimental.pallas import tpu as pltpu
from jax.experimental.pallas import tpu_sc as plsc

BATCH = 8
N_CLS = 50
N_CTX = 12
D = 768
W = 128
K = D // W
CTX_LEN = 77
N_TAIL = 64
MIX = 16
HEAD = 1 + N_CTX
TILES_PER_B = 4
CLS_PER_TILE = 13
LANES = 16
BLOCK = CTX_LEN * K


def _row6_copy(dst, drow, src, srow):
    for r in range(K):
        for c in range(0, W, LANES):
            dst[drow + r, pl.ds(c, LANES)] = src[srow + r, pl.ds(c, LANES)]


def _wide_to_row6(dst, drow, src, srow):
    for c in range(0, D, LANES):
        dst[drow + c // W, pl.ds(c % W, LANES)] = src[srow, pl.ds(c, LANES)]


def _body(ctx_hbm, pre_hbm, suf_hbm, emb_hbm, tok_hbm, nl_hbm, out_hbm,
          idx_v, ens_v, nl_v, pre_v, hd_v, mix_v, fet_v,
          sem_g, sem_s, sem_m0, sem_m1, sem_h0, sem_h1, sem_w):
    nc = 2
    wid = lax.axis_index("s") * nc + lax.axis_index("c")
    b = wid // TILES_PER_B
    g = wid % TILES_PER_B
    i_start = g * CLS_PER_TILE
    i_count = jnp.minimum(CLS_PER_TILE, N_CLS - i_start)
    sem_m = (sem_m0, sem_m1)
    sem_h = (sem_h0, sem_h1)

    pltpu.sync_copy(tok_hbm.at[b], idx_v)
    pltpu.sync_copy(ctx_hbm.at[b], hd_v.at[pl.ds(K, N_CTX * K)])
    pltpu.sync_copy(ctx_hbm.at[b], hd_v.at[pl.ds(HEAD * K + K, N_CTX * K)])
    pltpu.sync_copy(pre_hbm.at[pl.ds(i_start * K, CLS_PER_TILE * K)], pre_v)
    pltpu.sync_copy(nl_hbm, nl_v)

    for k in range(N_TAIL // MIX):
        pltpu.async_copy(
            emb_hbm.at[idx_v.at[pl.ds(k * MIX, MIX)]],
            fet_v.at[pl.ds(0, MIX)], sem_g).wait()
        for r in range(MIX):
            _wide_to_row6(ens_v, (k * MIX + r) * K, fet_v, r)

    def suf_fetch(j, par):
        return pltpu.make_async_copy(
            suf_hbm.at[i_start + j, pl.ds(0, MIX)],
            fet_v.at[pl.ds(par * MIX, MIX)], sem_s)

    def mix_write(j, par):
        base = (b * N_CLS + i_start + j) * BLOCK
        return pltpu.make_async_copy(
            mix_v.at[pl.ds(par * MIX * K, MIX * K)],
            out_hbm.at[pl.ds(base + HEAD * K, MIX * K)], sem_m[par])

    def head_write(j, par):
        base = (b * N_CLS + i_start + j) * BLOCK
        return pltpu.make_async_copy(
            hd_v.at[pl.ds(par * HEAD * K, HEAD * K)],
            out_hbm.at[pl.ds(base, HEAD * K)], sem_h[par])

    suf_fetch(0, 0).start()
    suf_fetch(1, 1).start()

    def process(j, par):
        i = i_start + j
        base = (b * N_CLS + i) * BLOCK
        ell = nl_v[pl.ds(i, 16)][0]
        suf_fetch(j, par).wait()

        @pl.when(j >= 2)
        def _():
            mix_write(j, par).wait()

        def suf_row(p, carry):
            _wide_to_row6(mix_v, par * MIX * K + p * K, fet_v, par * MIX + p)
            return carry

        def ens_row(p, carry):
            _row6_copy(mix_v, par * MIX * K + p * K, ens_v, (p - ell) * K)
            return carry

        lax.fori_loop(0, ell, suf_row, 0)
        lax.fori_loop(ell, MIX, ens_row, 0)

        @pl.when(j + 2 < i_count)
        def _():
            suf_fetch(j + 2, par).start()

        @pl.when(j >= 2)
        def _():
            head_write(j, par).wait()

        _row6_copy(hd_v, par * HEAD * K, pre_v, j * K)

        head_write(j, par).start()
        mix_write(j, par).start()
        pltpu.make_async_copy(
            ens_v.at[pl.ds((MIX - ell) * K, (N_TAIL - MIX) * K)],
            out_hbm.at[pl.ds(base + (HEAD + MIX) * K, (N_TAIL - MIX) * K)],
            sem_w).start()

    def outer(k, carry):
        j0 = 2 * k

        @pl.when(j0 < i_count)
        def _():
            process(j0, 0)

        @pl.when(j0 + 1 < i_count)
        def _():
            process(j0 + 1, 1)

        return carry

    lax.fori_loop(0, (CLS_PER_TILE + 1) // 2, outer, 0)

    mix_write(0, 0).wait()
    mix_write(0, 1).wait()
    head_write(0, 0).wait()
    head_write(0, 1).wait()

    def drain(j, carry):
        base = (b * N_CLS + i_start + j) * BLOCK
        pltpu.make_async_copy(
            ens_v.at[pl.ds(0, (N_TAIL - MIX) * K)],
            out_hbm.at[pl.ds(base + (HEAD + MIX) * K, (N_TAIL - MIX) * K)],
            sem_w).wait()
        return carry

    lax.fori_loop(0, i_count, drain, 0)


def kernel(ctx, token_prefix, token_suffix, embed_table, tokenized_ensemble,
           name_lens, tokenized_prompts):
    tok64 = tokenized_ensemble[:, :N_TAIL]
    ctx6 = ctx.reshape(BATCH, N_CTX * K, W)
    pre6 = jnp.zeros(((N_CLS + CLS_PER_TILE) * K, W), jnp.float32)
    pre6 = pre6.at[:N_CLS * K].set(token_prefix.reshape(N_CLS * K, W))
    nl64 = jnp.zeros((80,), jnp.int32).at[:N_CLS].set(name_lens)

    mesh = plsc.VectorSubcoreMesh(core_axis_name="c", subcore_axis_name="s")
    call = functools.partial(
        pl.kernel,
        mesh=mesh,
        compiler_params=pltpu.CompilerParams(use_tc_tiling_on_sc=False),
        out_type=jax.ShapeDtypeStruct((BATCH * N_CLS * BLOCK, W), jnp.float32),
        scratch_types=[
            pltpu.VMEM((N_TAIL,), jnp.int32),
            pltpu.VMEM((N_TAIL * K, W), jnp.float32),
            pltpu.VMEM((80,), jnp.int32),
            pltpu.VMEM((CLS_PER_TILE * K, W), jnp.float32),
            pltpu.VMEM((2 * HEAD * K, W), jnp.float32),
            pltpu.VMEM((2 * MIX * K, W), jnp.float32),
            pltpu.VMEM((2 * MIX, D), jnp.float32),
            pltpu.SemaphoreType.DMA,
            pltpu.SemaphoreType.DMA,
            pltpu.SemaphoreType.DMA,
            pltpu.SemaphoreType.DMA,
            pltpu.SemaphoreType.DMA,
            pltpu.SemaphoreType.DMA,
            pltpu.SemaphoreType.DMA,
        ],
    )(_body)
    flat = call(ctx6, pre6, token_suffix, embed_table, tok64, nl64)

    prompts = flat.reshape(BATCH * N_CLS, CTX_LEN, D)
    tp = jnp.broadcast_to(tokenized_prompts[None],
                          (BATCH, N_CLS, CTX_LEN)).reshape(BATCH * N_CLS, CTX_LEN)
    return (prompts, tp)


# speedup vs baseline: 1.6539x; 1.6539x over previous
import functools

import jax
import jax.numpy as jnp
from jax import lax
from jax.experimental import pallas as pl
from jax.experimental.pallas import tpu as pltpu
from jax.experimental.pallas import tpu_sc as plsc

BATCH = 8
N_CLS = 50
N_CTX = 12
D = 768
W = 128
K = D // W
CTX_LEN = 77
N_TAIL = 64
MIX = 16
HEAD = 1 + N_CTX
TILES_PER_B = 4
CLS_PER_TILE = 13
LANES = 16
BLOCK = CTX_LEN * K
N_TOK = BATCH * N_TAIL
TOK_PER_TILE = N_TOK // 32


def _gather_body(emb_hbm, idx_hbm, out_hbm, idx_v, row_v, sem):
    wid = lax.axis_index("s") * 2 + lax.axis_index("c")
    base = wid * TOK_PER_TILE
    pltpu.sync_copy(idx_hbm.at[pl.ds(base, TOK_PER_TILE)], idx_v)
    pltpu.async_copy(emb_hbm.at[idx_v], row_v, sem).wait()
    pltpu.sync_copy(row_v, out_hbm.at[pl.ds(base, TOK_PER_TILE)])


def _row6_copy(dst, drow, src, srow):
    for r in range(K):
        for c in range(0, W, LANES):
            dst[drow + r, pl.ds(c, LANES)] = src[srow + r, pl.ds(c, LANES)]


def _body(ctx_hbm, pre_hbm, suf_hbm, ens_hbm, nl_hbm, out_hbm,
          ens_v, nl_v, pre_v, hd_v, mix_v, fet_v,
          sem_s, sem_m0, sem_m1, sem_h0, sem_h1, sem_w):
    nc = 2
    wid = lax.axis_index("s") * nc + lax.axis_index("c")
    b = wid // TILES_PER_B
    g = wid % TILES_PER_B
    i_start = g * CLS_PER_TILE
    i_count = jnp.minimum(CLS_PER_TILE, N_CLS - i_start)
    sem_m = (sem_m0, sem_m1)
    sem_h = (sem_h0, sem_h1)

    pltpu.sync_copy(ens_hbm.at[pl.ds(b * N_TAIL * K, N_TAIL * K)], ens_v)
    pltpu.sync_copy(ctx_hbm.at[b], hd_v.at[pl.ds(K, N_CTX * K)])
    pltpu.sync_copy(ctx_hbm.at[b], hd_v.at[pl.ds(HEAD * K + K, N_CTX * K)])
    pltpu.sync_copy(pre_hbm.at[pl.ds(i_start * K, CLS_PER_TILE * K)], pre_v)
    pltpu.sync_copy(nl_hbm, nl_v)

    def suf_fetch(j, par):
        return pltpu.make_async_copy(
            suf_hbm.at[i_start + j], fet_v.at[pl.ds(par * MIX * K, MIX * K)],
            sem_s)

    def mix_write(j, par):
        base = (b * N_CLS + i_start + j) * BLOCK
        return pltpu.make_async_copy(
            mix_v.at[pl.ds(par * MIX * K, MIX * K)],
            out_hbm.at[pl.ds(base + HEAD * K, MIX * K)], sem_m[par])

    def head_write(j, par):
        base = (b * N_CLS + i_start + j) * BLOCK
        return pltpu.make_async_copy(
            hd_v.at[pl.ds(par * HEAD * K, HEAD * K)],
            out_hbm.at[pl.ds(base, HEAD * K)], sem_h[par])

    suf_fetch(0, 0).start()
    suf_fetch(1, 1).start()

    def process(j, par):
        i = i_start + j
        base = (b * N_CLS + i) * BLOCK
        ell = nl_v[pl.ds(i, 16)][0]
        suf_fetch(j, par).wait()

        @pl.when(j >= 2)
        def _():
            mix_write(j, par).wait()

        def suf_row(p, carry):
            _row6_copy(mix_v, par * MIX * K + p * K, fet_v,
                       par * MIX * K + p * K)
            return carry

        def ens_row(p, carry):
            _row6_copy(mix_v, par * MIX * K + p * K, ens_v, (p - ell) * K)
            return carry

        lax.fori_loop(0, ell, suf_row, 0)
        lax.fori_loop(ell, MIX, ens_row, 0)

        @pl.when(j + 2 < i_count)
        def _():
            suf_fetch(j + 2, par).start()

        @pl.when(j >= 2)
        def _():
            head_write(j, par).wait()

        _row6_copy(hd_v, par * HEAD * K, pre_v, j * K)

        head_write(j, par).start()
        mix_write(j, par).start()
        pltpu.make_async_copy(
            ens_v.at[pl.ds((MIX - ell) * K, (N_TAIL - MIX) * K)],
            out_hbm.at[pl.ds(base + (HEAD + MIX) * K, (N_TAIL - MIX) * K)],
            sem_w).start()

    def outer(k, carry):
        j0 = 2 * k

        @pl.when(j0 < i_count)
        def _():
            process(j0, 0)

        @pl.when(j0 + 1 < i_count)
        def _():
            process(j0 + 1, 1)

        return carry

    lax.fori_loop(0, (CLS_PER_TILE + 1) // 2, outer, 0)

    mix_write(0, 0).wait()
    mix_write(0, 1).wait()
    head_write(0, 0).wait()
    head_write(0, 1).wait()

    def drain(j, carry):
        base = (b * N_CLS + i_start + j) * BLOCK
        pltpu.make_async_copy(
            ens_v.at[pl.ds(0, (N_TAIL - MIX) * K)],
            out_hbm.at[pl.ds(base + (HEAD + MIX) * K, (N_TAIL - MIX) * K)],
            sem_w).wait()
        return carry

    lax.fori_loop(0, i_count, drain, 0)


def kernel(ctx, token_prefix, token_suffix, embed_table, tokenized_ensemble,
           name_lens, tokenized_prompts):
    mesh = plsc.VectorSubcoreMesh(core_axis_name="c", subcore_axis_name="s")

    idx1d = tokenized_ensemble[:, :N_TAIL].reshape(N_TOK)
    gather_call = functools.partial(
        pl.kernel,
        mesh=mesh,
        compiler_params=pltpu.CompilerParams(use_tc_tiling_on_sc=True),
        out_type=jax.ShapeDtypeStruct((N_TOK, D), jnp.float32),
        scratch_types=[
            pltpu.VMEM((TOK_PER_TILE,), jnp.int32),
            pltpu.VMEM((TOK_PER_TILE, D), jnp.float32),
            pltpu.SemaphoreType.DMA,
        ],
    )(_gather_body)
    ens_all = gather_call(embed_table, idx1d)

    ens6 = ens_all.reshape(N_TOK * K, W)
    suf6 = token_suffix[:, :MIX, :].reshape(N_CLS, MIX * K, W)
    ctx6 = ctx.reshape(BATCH, N_CTX * K, W)
    pre6 = jnp.zeros(((N_CLS + CLS_PER_TILE + 1) * K, W), jnp.float32)
    pre6 = pre6.at[:N_CLS * K].set(token_prefix.reshape(N_CLS * K, W))
    nl128 = jnp.zeros((128,), jnp.int32).at[:N_CLS].set(name_lens)

    call = functools.partial(
        pl.kernel,
        mesh=mesh,
        compiler_params=pltpu.CompilerParams(use_tc_tiling_on_sc=False),
        out_type=jax.ShapeDtypeStruct((BATCH * N_CLS * BLOCK, W), jnp.float32),
        scratch_types=[
            pltpu.VMEM((N_TAIL * K, W), jnp.float32),
            pltpu.VMEM((128,), jnp.int32),
            pltpu.VMEM((CLS_PER_TILE * K, W), jnp.float32),
            pltpu.VMEM((2 * HEAD * K, W), jnp.float32),
            pltpu.VMEM((2 * MIX * K, W), jnp.float32),
            pltpu.VMEM((2 * MIX * K, W), jnp.float32),
            pltpu.SemaphoreType.DMA,
            pltpu.SemaphoreType.DMA,
            pltpu.SemaphoreType.DMA,
            pltpu.SemaphoreType.DMA,
            pltpu.SemaphoreType.DMA,
            pltpu.SemaphoreType.DMA,
        ],
    )(_body)
    flat = call(ctx6, pre6, suf6, ens6, nl128)

    prompts = flat.reshape(BATCH * N_CLS, CTX_LEN, D)
    tp = jnp.broadcast_to(tokenized_prompts[None],
                          (BATCH, N_CLS, CTX_LEN)).reshape(BATCH * N_CLS, CTX_LEN)
    return (prompts, tp)
